# baseline (speedup 1.0000x reference)
import functools

import jax
import jax.numpy as jnp
from jax.experimental import pallas as pl
from jax.experimental.pallas import tpu as pltpu

D = 32
G4 = 4 * D
N_MLP = 2
N_ROUNDS = 4
FB = 1.0
EPS = 1e-5
NB = 16


def _relu_mlp(x, Ws, bs):
    for l in range(N_MLP):
        x = jnp.dot(Ws[l], x, preferred_element_type=jnp.float32,
                     precision=jax.lax.Precision.HIGHEST)
        x = jnp.maximum(x + bs[l], 0.0)
    return x


def _gated_update(pre, c, gamma, beta, gc, bc, lnmat, fbias, dmat):
    mu = jnp.dot(lnmat, pre, preferred_element_type=jnp.float32,
                     precision=jax.lax.Precision.HIGHEST)
    d = pre - mu
    var = jnp.dot(lnmat, d * d, preferred_element_type=jnp.float32,
                     precision=jax.lax.Precision.HIGHEST)
    z = d * jax.lax.rsqrt(var + EPS) * gamma + beta
    sg = jax.nn.sigmoid(z + fbias)
    g = jnp.tanh(z[2 * D:3 * D])
    c_new = sg[D:2 * D] * c + sg[0:D] * g
    mu2 = jnp.dot(dmat, c_new, preferred_element_type=jnp.float32,
                     precision=jax.lax.Precision.HIGHEST)
    d2 = c_new - mu2
    v2 = jnp.dot(dmat, d2 * d2, preferred_element_type=jnp.float32,
                     precision=jax.lax.Precision.HIGHEST)
    h = jnp.tanh(d2 * jax.lax.rsqrt(v2 + EPS) * gc + bc) * sg[3 * D:4 * D]
    return h, c_new


def _msgpass_kernel(cnt_ref, adj_ref,
                    lpos_ref, lneg_ref, cinit_ref,
                    lcW_ref, lcb_ref, clW_ref, clb_ref,
                    cwih_ref, cwhh_ref, cb_ref, cg_ref, cbe_ref, cgc_ref, cbc_ref,
                    lwihm_ref, lwihf_ref, lwhh_ref, lb_ref, lg_ref, lbe_ref, lgc_ref, lbc_ref,
                    rankw_ref, out_ref, *, nb):
    g0 = pl.program_id(0) * nb
    _, L, C = adj_ref.shape
    halfL = L // 2

    ns = [cnt_ref[g0 + i] for i in range(nb)]
    halves = [jax.lax.div(n, jnp.int32(2)) for n in ns]
    adjs = [adj_ref[i] for i in range(nb)]

    r128 = jax.lax.broadcasted_iota(jnp.int32, (G4, G4), 0)
    c128 = jax.lax.broadcasted_iota(jnp.int32, (G4, G4), 1)
    lnmat = jnp.where((r128 // D) == (c128 // D), 1.0 / D, 0.0)
    dmat = jnp.full((D, D), 1.0 / D, jnp.float32)
    srow = jax.lax.broadcasted_iota(jnp.int32, (G4, 1), 0)
    fbias = jnp.where((srow >= D) & (srow < 2 * D), FB, 0.0)

    colL = jax.lax.broadcasted_iota(jnp.int32, (D, nb * L), 1)
    L_h = jnp.where((colL % L) < halfL,
                    jnp.broadcast_to(lpos_ref[...], (D, nb * L)),
                    jnp.broadcast_to(lneg_ref[...], (D, nb * L)))
    C_h = jnp.broadcast_to(cinit_ref[...], (D, nb * C))
    L_c = jnp.zeros((D, nb * L), jnp.float32)
    C_c = jnp.zeros((D, nb * C), jnp.float32)

    lcW = lcW_ref[...]; lcb = lcb_ref[...]
    clW = clW_ref[...]; clb = clb_ref[...]
    cwih = cwih_ref[...]; cwhh = cwhh_ref[...]; cbias = cb_ref[...]
    cgam = cg_ref[...]; cbet = cbe_ref[...]; cgc = cgc_ref[...]; cbc = cbc_ref[...]
    lwihm = lwihm_ref[...]; lwihf = lwihf_ref[...]; lwhh = lwhh_ref[...]
    lbias = lb_ref[...]
    lgam = lg_ref[...]; lbet = lbe_ref[...]; lgc = lgc_ref[...]; lbc = lbc_ref[...]

    cc = jax.lax.broadcasted_iota(jnp.int32, (D, L), 1)

    for _ in range(N_ROUNDS):
        mL = _relu_mlp(L_h, lcW, lcb)
        lc = jnp.concatenate(
            [jnp.dot(mL[:, i * L:(i + 1) * L], adjs[i],
                     preferred_element_type=jnp.float32,
                     precision=jax.lax.Precision.HIGHEST)
             for i in range(nb)], axis=1)
        pre_c = (jnp.dot(cwih, lc, preferred_element_type=jnp.float32,
                     precision=jax.lax.Precision.HIGHEST)
                 + jnp.dot(cwhh, C_h, preferred_element_type=jnp.float32,
                     precision=jax.lax.Precision.HIGHEST)
                 + cbias)
        C_h, C_c = _gated_update(pre_c, C_c, cgam, cbet, cgc, cbc,
                                 lnmat, fbias, dmat)

        mC = _relu_mlp(C_h, clW, clb)
        cl = jnp.concatenate(
            [jax.lax.dot_general(mC[:, i * C:(i + 1) * C], adjs[i],
                                 (((1,), (1,)), ((), ())),
                                 preferred_element_type=jnp.float32,
                     precision=jax.lax.Precision.HIGHEST)
             for i in range(nb)], axis=1)

        flips = []
        for i in range(nb):
            lh_i = L_h[:, i * L:(i + 1) * L]
            dn = pltpu.roll(lh_i, -halves[i], axis=1)
            up = pltpu.roll(lh_i, halves[i], axis=1)
            flips.append(jnp.where(cc < halves[i], dn,
                                   jnp.where(cc < ns[i], up, 0.0)))
        flipped = jnp.concatenate(flips, axis=1)

        pre_l = (jnp.dot(lwihm, cl, preferred_element_type=jnp.float32,
                     precision=jax.lax.Precision.HIGHEST)
                 + jnp.dot(lwihf, flipped, preferred_element_type=jnp.float32,
                     precision=jax.lax.Precision.HIGHEST)
                 + jnp.dot(lwhh, L_h, preferred_element_type=jnp.float32,
                     precision=jax.lax.Precision.HIGHEST)
                 + lbias)
        L_h, L_c = _gated_update(pre_l, L_c, lgam, lbet, lgc, lbc,
                                 lnmat, fbias, dmat)

    w = rankw_ref[...]
    rh = jax.lax.broadcasted_iota(jnp.int32, (halfL, 1), 0)
    col2 = jax.lax.broadcasted_iota(jnp.int32, (halfL, 2), 1)
    for i in range(nb):
        top = L_h[:, i * L:i * L + halfL]
        s = jax.lax.dot_general(top, w, (((0,), (1,)), ((), ())),
                                preferred_element_type=jnp.float32,
                     precision=jax.lax.Precision.HIGHEST)
        m = (rh < halves[i]).astype(jnp.float32)
        out_ref[i] = jnp.where(col2 == 0, jnp.tanh(0.5 * s) * m, s * m)


def kernel(adjacency, batch_lit_counts, L_pos_init, L_neg_init, C_init,
           lc_W, lc_b, cl_W, cl_b,
           C_wih, C_whh, C_bias, C_gamma, C_beta, C_gc, C_bc,
           L_wih, L_whh, L_bias, L_gamma, L_beta, L_gc, L_bc, rank_w):
    B, L, C = adjacency.shape
    nb = NB
    while B % nb:
        nb //= 2
    counts = jnp.asarray(batch_lit_counts, jnp.int32)

    tv = lambda v: v.T
    tm = lambda m: m.T
    args = (adjacency,
            tv(L_pos_init), tv(L_neg_init), tv(C_init),
            jnp.transpose(lc_W, (0, 2, 1)), lc_b[:, :, None],
            jnp.transpose(cl_W, (0, 2, 1)), cl_b[:, :, None],
            tm(C_wih), tm(C_whh), tv(C_bias), tv(C_gamma), tv(C_beta),
            tv(C_gc), tv(C_bc),
            tm(L_wih[:D]), tm(L_wih[D:2 * D]), tm(L_whh), tv(L_bias),
            tv(L_gamma), tv(L_beta), tv(L_gc), tv(L_bc),
            rank_w)

    def whole(a):
        nd = a.ndim
        return pl.BlockSpec(a.shape, lambda b, cnt, _nd=nd: (0,) * _nd)

    in_specs = ([pl.BlockSpec((nb, L, C), lambda b, cnt: (b, 0, 0))]
                + [whole(a) for a in args[1:]])

    out = pl.pallas_call(
        functools.partial(_msgpass_kernel, nb=nb),
        out_shape=jax.ShapeDtypeStruct((B, L // 2, 2), jnp.float32),
        grid_spec=pltpu.PrefetchScalarGridSpec(
            num_scalar_prefetch=1,
            grid=(B // nb,),
            in_specs=in_specs,
            out_specs=pl.BlockSpec((nb, L // 2, 2), lambda b, cnt: (b, 0, 0)),
        ),
        compiler_params=pltpu.CompilerParams(dimension_semantics=("parallel",)),
    )(counts, *args)

    return out[:, :, 0:1], out[:, :, 1:2]


# speedup vs baseline: 1.1578x; 1.1578x over previous
import functools

import jax
import jax.numpy as jnp
from jax.experimental import pallas as pl
from jax.experimental.pallas import tpu as pltpu

D = 32
G4 = 4 * D
N_MLP = 2
N_ROUNDS = 4
FB = 1.0
EPS = 1e-5
NB = 16


def _relu_mlp(x, Ws, bs):
    for l in range(N_MLP):
        x = jnp.dot(Ws[l], x, preferred_element_type=jnp.float32,
                     precision=jax.lax.Precision.HIGHEST)
        x = jnp.maximum(x + bs[l], 0.0)
    return x


def _gated_update(pre, c, gamma, beta, gc, bc, lnmat, fbias, dmat):
    mu = jnp.dot(lnmat, pre, preferred_element_type=jnp.float32,
                     precision=jax.lax.Precision.HIGHEST)
    d = pre - mu
    var = jnp.dot(lnmat, d * d, preferred_element_type=jnp.float32,
                     precision=jax.lax.Precision.HIGHEST)
    z = d * jax.lax.rsqrt(var + EPS) * gamma + beta
    sg = jax.nn.sigmoid(z + fbias)
    g = jnp.tanh(z[2 * D:3 * D])
    c_new = sg[D:2 * D] * c + sg[0:D] * g
    mu2 = jnp.dot(dmat, c_new, preferred_element_type=jnp.float32,
                     precision=jax.lax.Precision.HIGHEST)
    d2 = c_new - mu2
    v2 = jnp.dot(dmat, d2 * d2, preferred_element_type=jnp.float32,
                     precision=jax.lax.Precision.HIGHEST)
    h = jnp.tanh(d2 * jax.lax.rsqrt(v2 + EPS) * gc + bc) * sg[3 * D:4 * D]
    return h, c_new


def _msgpass_kernel(cnt_ref, adj_ref,
                    lpos_ref, lneg_ref, cinit_ref,
                    lcW_ref, lcb_ref, clW_ref, clb_ref,
                    cwih_ref, cwhh_ref, cb_ref, cg_ref, cbe_ref, cgc_ref, cbc_ref,
                    lwihm_ref, lwihf_ref, lwhh_ref, lb_ref, lg_ref, lbe_ref, lgc_ref, lbc_ref,
                    rankw_ref, out_ref, *, nb):
    g0 = pl.program_id(0) * nb
    _, L, C = adj_ref.shape
    halfL = L // 2

    ns = [cnt_ref[g0 + i] for i in range(nb)]
    halves = [jax.lax.div(n, jnp.int32(2)) for n in ns]
    adjs = [adj_ref[i] for i in range(nb)]

    r128 = jax.lax.broadcasted_iota(jnp.int32, (G4, G4), 0)
    c128 = jax.lax.broadcasted_iota(jnp.int32, (G4, G4), 1)
    lnmat = jnp.where((r128 // D) == (c128 // D), 1.0 / D, 0.0)
    dmat = jnp.full((D, D), 1.0 / D, jnp.float32)
    srow = jax.lax.broadcasted_iota(jnp.int32, (G4, 1), 0)
    fbias = jnp.where((srow >= D) & (srow < 2 * D), FB, 0.0)

    colL = jax.lax.broadcasted_iota(jnp.int32, (D, nb * L), 1)
    L_h = jnp.where((colL % L) < halfL,
                    jnp.broadcast_to(lpos_ref[...], (D, nb * L)),
                    jnp.broadcast_to(lneg_ref[...], (D, nb * L)))
    C_h = jnp.broadcast_to(cinit_ref[...], (D, nb * C))
    L_c = jnp.zeros((D, nb * L), jnp.float32)
    C_c = jnp.zeros((D, nb * C), jnp.float32)

    lcW = lcW_ref[...]; lcb = lcb_ref[...]
    clW = clW_ref[...]; clb = clb_ref[...]
    cwih = cwih_ref[...]; cwhh = cwhh_ref[...]; cbias = cb_ref[...]
    cgam = cg_ref[...]; cbet = cbe_ref[...]; cgc = cgc_ref[...]; cbc = cbc_ref[...]
    lwihm = lwihm_ref[...]; lwihf = lwihf_ref[...]; lwhh = lwhh_ref[...]
    lbias = lb_ref[...]
    lgam = lg_ref[...]; lbet = lbe_ref[...]; lgc = lgc_ref[...]; lbc = lbc_ref[...]

    cc = jax.lax.broadcasted_iota(jnp.int32, (D, L), 1)

    for _ in range(N_ROUNDS):
        mL = _relu_mlp(L_h, lcW, lcb)
        mLh = mL.astype(jnp.bfloat16)
        mLl = (mL - mLh.astype(jnp.float32)).astype(jnp.bfloat16)
        lc = jnp.concatenate(
            [jnp.dot(mLh[:, i * L:(i + 1) * L], adjs[i],
                     preferred_element_type=jnp.float32)
             + jnp.dot(mLl[:, i * L:(i + 1) * L], adjs[i],
                       preferred_element_type=jnp.float32)
             for i in range(nb)], axis=1)
        pre_c = (jnp.dot(cwih, lc, preferred_element_type=jnp.float32,
                     precision=jax.lax.Precision.HIGHEST)
                 + jnp.dot(cwhh, C_h, preferred_element_type=jnp.float32,
                     precision=jax.lax.Precision.HIGHEST)
                 + cbias)
        C_h, C_c = _gated_update(pre_c, C_c, cgam, cbet, cgc, cbc,
                                 lnmat, fbias, dmat)

        mC = _relu_mlp(C_h, clW, clb)
        mCh = mC.astype(jnp.bfloat16)
        mCl = (mC - mCh.astype(jnp.float32)).astype(jnp.bfloat16)
        cl = jnp.concatenate(
            [jax.lax.dot_general(mCh[:, i * C:(i + 1) * C], adjs[i],
                                 (((1,), (1,)), ((), ())),
                                 preferred_element_type=jnp.float32)
             + jax.lax.dot_general(mCl[:, i * C:(i + 1) * C], adjs[i],
                                   (((1,), (1,)), ((), ())),
                                   preferred_element_type=jnp.float32)
             for i in range(nb)], axis=1)

        flips = []
        for i in range(nb):
            lh_i = L_h[:, i * L:(i + 1) * L]
            dn = pltpu.roll(lh_i, -halves[i], axis=1)
            up = pltpu.roll(lh_i, halves[i], axis=1)
            flips.append(jnp.where(cc < halves[i], dn,
                                   jnp.where(cc < ns[i], up, 0.0)))
        flipped = jnp.concatenate(flips, axis=1)

        pre_l = (jnp.dot(lwihm, cl, preferred_element_type=jnp.float32,
                     precision=jax.lax.Precision.HIGHEST)
                 + jnp.dot(lwihf, flipped, preferred_element_type=jnp.float32,
                     precision=jax.lax.Precision.HIGHEST)
                 + jnp.dot(lwhh, L_h, preferred_element_type=jnp.float32,
                     precision=jax.lax.Precision.HIGHEST)
                 + lbias)
        L_h, L_c = _gated_update(pre_l, L_c, lgam, lbet, lgc, lbc,
                                 lnmat, fbias, dmat)

    w = rankw_ref[...]
    rh = jax.lax.broadcasted_iota(jnp.int32, (halfL, 1), 0)
    col2 = jax.lax.broadcasted_iota(jnp.int32, (halfL, 2), 1)
    for i in range(nb):
        top = L_h[:, i * L:i * L + halfL]
        s = jax.lax.dot_general(top, w, (((0,), (1,)), ((), ())),
                                preferred_element_type=jnp.float32,
                     precision=jax.lax.Precision.HIGHEST)
        m = (rh < halves[i]).astype(jnp.float32)
        out_ref[i] = jnp.where(col2 == 0, jnp.tanh(0.5 * s) * m, s * m)


def kernel(adjacency, batch_lit_counts, L_pos_init, L_neg_init, C_init,
           lc_W, lc_b, cl_W, cl_b,
           C_wih, C_whh, C_bias, C_gamma, C_beta, C_gc, C_bc,
           L_wih, L_whh, L_bias, L_gamma, L_beta, L_gc, L_bc, rank_w):
    B, L, C = adjacency.shape
    nb = NB
    while B % nb:
        nb //= 2
    counts = jnp.asarray(batch_lit_counts, jnp.int32)
    adjacency = adjacency.astype(jnp.bfloat16)

    tv = lambda v: v.T
    tm = lambda m: m.T
    args = (adjacency,
            tv(L_pos_init), tv(L_neg_init), tv(C_init),
            jnp.transpose(lc_W, (0, 2, 1)), lc_b[:, :, None],
            jnp.transpose(cl_W, (0, 2, 1)), cl_b[:, :, None],
            tm(C_wih), tm(C_whh), tv(C_bias), tv(C_gamma), tv(C_beta),
            tv(C_gc), tv(C_bc),
            tm(L_wih[:D]), tm(L_wih[D:2 * D]), tm(L_whh), tv(L_bias),
            tv(L_gamma), tv(L_beta), tv(L_gc), tv(L_bc),
            rank_w)

    def whole(a):
        nd = a.ndim
        return pl.BlockSpec(a.shape, lambda b, cnt, _nd=nd: (0,) * _nd)

    in_specs = ([pl.BlockSpec((nb, L, C), lambda b, cnt: (b, 0, 0))]
                + [whole(a) for a in args[1:]])

    out = pl.pallas_call(
        functools.partial(_msgpass_kernel, nb=nb),
        out_shape=jax.ShapeDtypeStruct((B, L // 2, 2), jnp.float32),
        grid_spec=pltpu.PrefetchScalarGridSpec(
            num_scalar_prefetch=1,
            grid=(B // nb,),
            in_specs=in_specs,
            out_specs=pl.BlockSpec((nb, L // 2, 2), lambda b, cnt: (b, 0, 0)),
        ),
        compiler_params=pltpu.CompilerParams(dimension_semantics=("parallel",)),
    )(counts, *args)

    return out[:, :, 0:1], out[:, :, 1:2]


# speedup vs baseline: 1.2910x; 1.1151x over previous
import functools

import jax
import jax.numpy as jnp
from jax.experimental import pallas as pl
from jax.experimental.pallas import tpu as pltpu

D = 32
G4 = 4 * D
N_MLP = 2
N_ROUNDS = 4
FB = 1.0
EPS = 1e-5
NB = 16

_BF = jnp.bfloat16
_F = jnp.float32


def _split(x):
    h = x.astype(_BF)
    l = (x - h.astype(_F)).astype(_BF)
    return h, l


def _edot(a, x):
    h = x.astype(_BF)
    r = x - h.astype(_F)
    m = r.astype(_BF)
    l = (r - m.astype(_F)).astype(_BF)
    return (jnp.dot(a, h, preferred_element_type=_F)
            + jnp.dot(a, m, preferred_element_type=_F)
            + jnp.dot(a, l, preferred_element_type=_F))


def _hdot(w, x):
    return jnp.dot(w, x, preferred_element_type=_F,
                   precision=jax.lax.Precision.HIGHEST)


def _relu_mlp(x, Ws, bs):
    for l in range(N_MLP):
        x = jnp.maximum(_hdot(Ws[l], x) + bs[l], 0.0)
    return x


def _gated_update(pre, c, gamma, beta, gc, bc, lnmat, fbias, dmat):
    mu = _edot(lnmat, pre)
    d = pre - mu
    var = _edot(lnmat, d * d)
    z = d * jax.lax.rsqrt(var + EPS) * gamma + beta
    sg = jax.nn.sigmoid(z + fbias)
    g = jnp.tanh(z[2 * D:3 * D])
    c_new = sg[D:2 * D] * c + sg[0:D] * g
    mu2 = _edot(dmat, c_new)
    d2 = c_new - mu2
    v2 = _edot(dmat, d2 * d2)
    h = jnp.tanh(d2 * jax.lax.rsqrt(v2 + EPS) * gc + bc) * sg[3 * D:4 * D]
    return h, c_new


def _msgpass_kernel(cnt_ref, adj_ref,
                    lpos_ref, lneg_ref, cinit_ref,
                    lcW_ref, lcb_ref, clW_ref, clb_ref,
                    cwih_ref, cwhh_ref,
                    cb_ref, cg_ref, cbe_ref, cgc_ref, cbc_ref,
                    lwm_ref, lwf_ref, lwhh_ref,
                    lb_ref, lg_ref, lbe_ref, lgc_ref, lbc_ref,
                    rankw_ref, out_ref, *, nb):
    g0 = pl.program_id(0) * nb
    _, L, C = adj_ref.shape
    halfL = L // 2

    ns = [cnt_ref[g0 + i] for i in range(nb)]
    halves = [jax.lax.div(n, jnp.int32(2)) for n in ns]
    adjs = [adj_ref[i] for i in range(nb)]

    r128 = jax.lax.broadcasted_iota(jnp.int32, (G4, G4), 0)
    c128 = jax.lax.broadcasted_iota(jnp.int32, (G4, G4), 1)
    lnmat = jnp.where((r128 // D) == (c128 // D), 1.0 / D, 0.0).astype(_BF)
    dmat = jnp.full((D, D), 1.0 / D, _BF)
    srow = jax.lax.broadcasted_iota(jnp.int32, (G4, 1), 0)
    fbias = jnp.where((srow >= D) & (srow < 2 * D), FB, 0.0)

    colL = jax.lax.broadcasted_iota(jnp.int32, (D, nb * L), 1)
    L_h = jnp.where((colL % L) < halfL,
                    jnp.broadcast_to(lpos_ref[...], (D, nb * L)),
                    jnp.broadcast_to(lneg_ref[...], (D, nb * L)))
    C_h = jnp.broadcast_to(cinit_ref[...], (D, nb * C))
    L_c = jnp.zeros((D, nb * L), _F)
    C_c = jnp.zeros((D, nb * C), _F)

    lcW = lcW_ref[...]; lcb = lcb_ref[...]
    clW = clW_ref[...]; clb = clb_ref[...]
    cwih = cwih_ref[...]; cwhh = cwhh_ref[...]
    cbias = cb_ref[...]
    cgam = cg_ref[...]; cbet = cbe_ref[...]; cgc = cgc_ref[...]; cbc = cbc_ref[...]
    lwm = lwm_ref[...]; lwf = lwf_ref[...]; lwhh = lwhh_ref[...]
    lbias = lb_ref[...]
    lgam = lg_ref[...]; lbet = lbe_ref[...]; lgc = lgc_ref[...]; lbc = lbc_ref[...]

    cc = jax.lax.broadcasted_iota(jnp.int32, (D, L), 1)

    for _ in range(N_ROUNDS):
        mLh, mLl = _split(_relu_mlp(L_h, lcW, lcb))
        lc = jnp.concatenate(
            [jnp.dot(mLh[:, i * L:(i + 1) * L], adjs[i],
                     preferred_element_type=_F)
             + jnp.dot(mLl[:, i * L:(i + 1) * L], adjs[i],
                       preferred_element_type=_F)
             for i in range(nb)], axis=1)
        pre_c = _hdot(cwih, lc) + _hdot(cwhh, C_h) + cbias
        C_h, C_c = _gated_update(pre_c, C_c, cgam, cbet, cgc, cbc,
                                 lnmat, fbias, dmat)

        mCh, mCl = _split(_relu_mlp(C_h, clW, clb))
        cl = jnp.concatenate(
            [jax.lax.dot_general(mCh[:, i * C:(i + 1) * C], adjs[i],
                                 (((1,), (1,)), ((), ())),
                                 preferred_element_type=_F)
             + jax.lax.dot_general(mCl[:, i * C:(i + 1) * C], adjs[i],
                                   (((1,), (1,)), ((), ())),
                                   preferred_element_type=_F)
             for i in range(nb)], axis=1)

        flips = []
        for i in range(nb):
            lh_i = L_h[:, i * L:(i + 1) * L]
            dn = pltpu.roll(lh_i, -halves[i], axis=1)
            up = pltpu.roll(lh_i, halves[i], axis=1)
            flips.append(jnp.where(cc < halves[i], dn,
                                   jnp.where(cc < ns[i], up, 0.0)))
        flipped = jnp.concatenate(flips, axis=1)

        pre_l = (_hdot(lwm, cl) + _hdot(lwf, flipped) + _hdot(lwhh, L_h)
                 + lbias)
        L_h, L_c = _gated_update(pre_l, L_c, lgam, lbet, lgc, lbc,
                                 lnmat, fbias, dmat)

    w = rankw_ref[...]
    rh = jax.lax.broadcasted_iota(jnp.int32, (halfL, 1), 0)
    col2 = jax.lax.broadcasted_iota(jnp.int32, (halfL, 2), 1)
    for i in range(nb):
        top = L_h[:, i * L:i * L + halfL]
        s = jax.lax.dot_general(top, w, (((0,), (1,)), ((), ())),
                                preferred_element_type=_F,
                                precision=jax.lax.Precision.HIGHEST)
        m = (rh < halves[i]).astype(_F)
        out_ref[i] = jnp.where(col2 == 0, jnp.tanh(0.5 * s) * m, s * m)


def kernel(adjacency, batch_lit_counts, L_pos_init, L_neg_init, C_init,
           lc_W, lc_b, cl_W, cl_b,
           C_wih, C_whh, C_bias, C_gamma, C_beta, C_gc, C_bc,
           L_wih, L_whh, L_bias, L_gamma, L_beta, L_gc, L_bc, rank_w):
    B, L, C = adjacency.shape
    nb = NB
    while B % nb:
        nb //= 2
    counts = jnp.asarray(batch_lit_counts, jnp.int32)
    adjacency = adjacency.astype(_BF)

    tv = lambda v: v.T
    tm = lambda m: m.T

    args = (adjacency,
            tv(L_pos_init), tv(L_neg_init), tv(C_init),
            jnp.transpose(lc_W, (0, 2, 1)), lc_b[:, :, None],
            jnp.transpose(cl_W, (0, 2, 1)), cl_b[:, :, None],
            tm(C_wih), tm(C_whh),
            tv(C_bias), tv(C_gamma), tv(C_beta), tv(C_gc), tv(C_bc),
            tm(L_wih[:D]), tm(L_wih[D:2 * D]), tm(L_whh),
            tv(L_bias), tv(L_gamma), tv(L_beta), tv(L_gc), tv(L_bc),
            rank_w)

    def whole(a):
        nd = a.ndim
        return pl.BlockSpec(a.shape, lambda b, cnt, _nd=nd: (0,) * _nd)

    in_specs = ([pl.BlockSpec((nb, L, C), lambda b, cnt: (b, 0, 0))]
                + [whole(a) for a in args[1:]])

    out = pl.pallas_call(
        functools.partial(_msgpass_kernel, nb=nb),
        out_shape=jax.ShapeDtypeStruct((B, L // 2, 2), jnp.float32),
        grid_spec=pltpu.PrefetchScalarGridSpec(
            num_scalar_prefetch=1,
            grid=(B // nb,),
            in_specs=in_specs,
            out_specs=pl.BlockSpec((nb, L // 2, 2), lambda b, cnt: (b, 0, 0)),
        ),
        compiler_params=pltpu.CompilerParams(dimension_semantics=("parallel",)),
    )(counts, *args)

    return out[:, :, 0:1], out[:, :, 1:2]
